# baseline (speedup 1.0000x reference)
import functools

import jax
import jax.numpy as jnp
from jax import lax
from jax.experimental import pallas as pl
from jax.experimental.pallas import tpu as pltpu
from jax.experimental.pallas import tpu_sc as plsc

N = 10000
E = 320000
NS = 16
K = 125
NCH = E // (NS * K)
ROWS_A = 624
TAIL = N - NS * ROWS_A


def _make_propagate(fsc, nbuf):
    mesh = plsc.VectorSubcoreMesh(core_axis_name="c", subcore_axis_name="s")

    @functools.partial(
        pl.kernel,
        mesh=mesh,
        compiler_params=pltpu.CompilerParams(use_tc_tiling_on_sc=False),
        out_type=jax.ShapeDtypeStruct((2 * N, fsc), jnp.float32),
        scratch_types=[
            pltpu.VMEM((NCH, K), jnp.int32),
            pltpu.VMEM((NCH, K), jnp.int32),
        ]
        + [pltpu.VMEM((K, fsc), jnp.float32) for _ in range(nbuf)]
        + [pltpu.VMEM_SHARED((N, fsc), jnp.float32)]
        + [pltpu.SemaphoreType.DMA for _ in range(nbuf)],
    )
    def prop(z_hbm, gsrc_hbm, gdst_hbm, out_hbm, src_v, dst_v, *rest):
        bufs = rest[:nbuf]
        acc = rest[nbuf]
        sems = rest[nbuf + 1:]
        c = lax.axis_index("c")
        s = lax.axis_index("s")
        pltpu.sync_copy(gsrc_hbm.at[c, s], src_v)
        pltpu.sync_copy(gdst_hbm.at[s], dst_v)
        pltpu.sync_copy(z_hbm.at[pl.ds(c * N + s * ROWS_A, ROWS_A)],
                        acc.at[pl.ds(s * ROWS_A, ROWS_A)])

        @pl.when(s == 0)
        def _():
            pltpu.sync_copy(z_hbm.at[pl.ds(c * N + NS * ROWS_A, TAIL)],
                            acc.at[pl.ds(NS * ROWS_A, TAIL)])

        plsc.subcore_barrier()

        def gather(j, b):
            pltpu.make_async_copy(z_hbm.at[src_v.at[j]], bufs[b], sems[b]).start()

        def gwait(b):
            pltpu.make_async_copy(z_hbm.at[src_v.at[0]], bufs[b], sems[b]).wait()

        for b in range(nbuf - 1):
            gather(b, b)

        def body(j0, carry):
            j = j0 * nbuf
            for b in range(nbuf):
                gwait(b)

                @pl.when(j + b + nbuf - 1 < NCH)
                def _():
                    gather(j + b + nbuf - 1, (b + nbuf - 1) % nbuf)

                pltpu.sync_copy(bufs[b], acc.at[dst_v.at[j + b]], add=True)
            return carry

        lax.fori_loop(0, NCH // nbuf, body, 0)
        plsc.subcore_barrier()
        pltpu.sync_copy(acc.at[pl.ds(s * ROWS_A, ROWS_A)],
                        out_hbm.at[pl.ds(c * N + s * ROWS_A, ROWS_A)])

        @pl.when(s == 0)
        def _():
            pltpu.sync_copy(acc.at[pl.ds(NS * ROWS_A, TAIL)],
                            out_hbm.at[pl.ds(c * N + NS * ROWS_A, TAIL)])

    return prop


def _make_layer2():
    mesh = plsc.VectorSubcoreMesh(core_axis_name="c", subcore_axis_name="s")
    nbuf = 4
    nbuf16 = 2

    @functools.partial(
        pl.kernel,
        mesh=mesh,
        compiler_params=pltpu.CompilerParams(use_tc_tiling_on_sc=False),
        out_type=(jax.ShapeDtypeStruct((6 * N, 64), jnp.float32),
                  jax.ShapeDtypeStruct((2 * N, 16), jnp.float32)),
        scratch_types=[
            pltpu.VMEM((NCH, K), jnp.int32),
            pltpu.VMEM((NCH, K), jnp.int32),
        ]
        + [pltpu.VMEM((K, 64), jnp.float32) for _ in range(nbuf)]
        + [pltpu.VMEM((K, 16), jnp.float32) for _ in range(nbuf16)]
        + [pltpu.VMEM_SHARED((N, 64), jnp.float32),
           pltpu.VMEM_SHARED((N, 16), jnp.float32)]
        + [pltpu.SemaphoreType.DMA for _ in range(nbuf)],
    )
    def l2(za_hbm, zb_hbm, zc_hbm, zt_hbm, gsrc_hbm, gdst_hbm,
           out6_hbm, outt_hbm, src_v, dst_v, *rest):
        bufs64 = rest[:nbuf]
        bufs16 = rest[nbuf:nbuf + nbuf16]
        acc64 = rest[nbuf + nbuf16]
        acc16 = rest[nbuf + nbuf16 + 1]
        sems = rest[nbuf + nbuf16 + 2:]
        c = lax.axis_index("c")
        s = lax.axis_index("s")
        pltpu.sync_copy(gsrc_hbm.at[c, s], src_v)
        pltpu.sync_copy(gdst_hbm.at[s], dst_v)

        def round_(q, z_hbm, out_hbm, bufs, acc, obase):
            nb = len(bufs)
            zbase = c * N
            pltpu.sync_copy(z_hbm.at[pl.ds(zbase + s * ROWS_A, ROWS_A)],
                            acc.at[pl.ds(s * ROWS_A, ROWS_A)])

            @pl.when(s == 0)
            def _():
                pltpu.sync_copy(z_hbm.at[pl.ds(zbase + NS * ROWS_A, TAIL)],
                                acc.at[pl.ds(NS * ROWS_A, TAIL)])

            plsc.subcore_barrier()

            def gather(j, b):
                pltpu.make_async_copy(z_hbm.at[src_v.at[j]], bufs[b],
                                      sems[b]).start()

            def gwait(b):
                pltpu.make_async_copy(z_hbm.at[src_v.at[0]], bufs[b],
                                      sems[b]).wait()

            for b in range(nb - 1):
                gather(b, b)

            def body(j0, carry):
                j = j0 * nb
                for b in range(nb):
                    gwait(b)

                    @pl.when(j + b + nb - 1 < NCH)
                    def _():
                        gather(j + b + nb - 1, (b + nb - 1) % nb)

                    pltpu.sync_copy(bufs[b], acc.at[dst_v.at[j + b]], add=True)
                return carry

            lax.fori_loop(0, NCH // nb, body, 0)
            plsc.subcore_barrier()
            pltpu.sync_copy(acc.at[pl.ds(s * ROWS_A, ROWS_A)],
                            out_hbm.at[pl.ds(obase + s * ROWS_A, ROWS_A)])

            @pl.when(s == 0)
            def _():
                pltpu.sync_copy(acc.at[pl.ds(NS * ROWS_A, TAIL)],
                                out_hbm.at[pl.ds(obase + NS * ROWS_A, TAIL)])

        for q, z_hbm in enumerate([za_hbm, zb_hbm, zc_hbm]):
            round_(q, z_hbm, out6_hbm, bufs64, acc64, (2 * q + c) * N)
        round_(3, zt_hbm, outt_hbm, bufs16, acc16, c * N)

    return l2


def _make_degree():
    mesh = plsc.VectorSubcoreMesh(core_axis_name="c", subcore_axis_name="s")

    @functools.partial(
        pl.kernel,
        mesh=mesh,
        compiler_params=pltpu.CompilerParams(use_tc_tiling_on_sc=False),
        out_type=jax.ShapeDtypeStruct((2 * N, 16), jnp.float32),
        scratch_types=[
            pltpu.VMEM((NCH, K), jnp.int32),
            pltpu.VMEM((K, 16), jnp.float32),
            pltpu.VMEM_SHARED((N, 16), jnp.float32),
        ],
    )
    def deg(ones_hbm, gdst_hbm, out_hbm, dst_v, ones_v, acc):
        c = lax.axis_index("c")
        s = lax.axis_index("s")
        pltpu.sync_copy(gdst_hbm.at[s], dst_v)
        pltpu.sync_copy(ones_hbm.at[pl.ds(0, K)], ones_v)
        pltpu.sync_copy(ones_hbm.at[pl.ds(s * ROWS_A, ROWS_A)],
                        acc.at[pl.ds(s * ROWS_A, ROWS_A)])

        @pl.when(s == 0)
        def _():
            pltpu.sync_copy(ones_hbm.at[pl.ds(NS * ROWS_A, TAIL)],
                            acc.at[pl.ds(NS * ROWS_A, TAIL)])

        plsc.subcore_barrier()

        def body(j, carry):
            pltpu.sync_copy(ones_v, acc.at[dst_v.at[j]], add=True)
            return carry

        lax.fori_loop(0, NCH, body, 0)
        plsc.subcore_barrier()
        pltpu.sync_copy(acc.at[pl.ds(s * ROWS_A, ROWS_A)],
                        out_hbm.at[pl.ds(c * N + s * ROWS_A, ROWS_A)])

        @pl.when(s == 0)
        def _():
            pltpu.sync_copy(acc.at[pl.ds(NS * ROWS_A, TAIL)],
                            out_hbm.at[pl.ds(c * N + NS * ROWS_A, TAIL)])

    return deg


_B = 1000


def _scale_body(x_ref, deg_ref, u_ref):
    dinv = lax.rsqrt(deg_ref[...])
    u = x_ref[...] * dinv
    u_ref[0] = u[:, :64]
    u_ref[1] = u[:, 64:]


def _scale_x(x, indeg):
    return pl.pallas_call(
        _scale_body,
        grid=(N // _B,),
        in_specs=[
            pl.BlockSpec((_B, 128), lambda i: (i, 0)),
            pl.BlockSpec((_B, 1), lambda i: (i, 0)),
        ],
        out_specs=pl.BlockSpec((2, _B, 64), lambda i: (0, i, 0)),
        out_shape=jax.ShapeDtypeStruct((2, N, 64), jnp.float32),
    )(x, indeg)


def _layer12_body(s1_ref, deg_ref, w1_ref, b1_ref, w2_ref, z2s_ref, z2t_ref):
    dinv = lax.rsqrt(deg_ref[...])
    s1 = jnp.concatenate([s1_ref[0], s1_ref[1]], axis=1)
    h = jnp.dot(s1 * dinv, w1_ref[...], preferred_element_type=jnp.float32)
    h = jnp.maximum(h + b1_ref[...], 0.0)
    z2 = jnp.dot(h, w2_ref[...], preferred_element_type=jnp.float32) * dinv
    for k in range(6):
        z2s_ref[k] = z2[:, k * 64:(k + 1) * 64]
    pad = jnp.zeros((z2.shape[0], 8), jnp.float32)
    z2t_ref[0] = jnp.concatenate([z2[:, 384:392], pad], axis=1)
    z2t_ref[1] = jnp.concatenate([z2[:, 392:400], pad], axis=1)


def _layer12(s1, indeg, W1, b1, W2):
    return pl.pallas_call(
        _layer12_body,
        grid=(N // _B,),
        in_specs=[
            pl.BlockSpec((2, _B, 64), lambda i: (0, i, 0)),
            pl.BlockSpec((_B, 1), lambda i: (i, 0)),
            pl.BlockSpec((128, 800), lambda i: (0, 0)),
            pl.BlockSpec((1, 800), lambda i: (0, 0)),
            pl.BlockSpec((800, 400), lambda i: (0, 0)),
        ],
        out_specs=[pl.BlockSpec((6, _B, 64), lambda i: (0, i, 0)),
                   pl.BlockSpec((2, _B, 16), lambda i: (0, i, 0))],
        out_shape=[jax.ShapeDtypeStruct((6, N, 64), jnp.float32),
                   jax.ShapeDtypeStruct((2, N, 16), jnp.float32)],
    )(s1, indeg, W1, b1, W2)


def _layer23_body(s2s_ref, s2t_ref, deg_ref, b2_ref, w3_ref, z3_ref):
    dinv = lax.rsqrt(deg_ref[...])
    s2 = jnp.concatenate([s2s_ref[k] for k in range(6)]
                         + [s2t_ref[0][:, 0:8], s2t_ref[1][:, 0:8]], axis=1)
    t2 = jnp.maximum(s2 * dinv + b2_ref[...], 0.0)
    z3 = jnp.dot(t2, w3_ref[...], preferred_element_type=jnp.float32) * dinv
    pad = jnp.zeros((z3.shape[0], 12), jnp.float32)
    z3_ref[0] = jnp.concatenate([z3[:, :4], pad], axis=1)
    z3_ref[1] = jnp.concatenate([z3[:, 4:], pad], axis=1)


def _layer23(s2s, s2t, indeg, b2, W3):
    return pl.pallas_call(
        _layer23_body,
        grid=(N // _B,),
        in_specs=[
            pl.BlockSpec((6, _B, 64), lambda i: (0, i, 0)),
            pl.BlockSpec((2, _B, 16), lambda i: (0, i, 0)),
            pl.BlockSpec((_B, 1), lambda i: (i, 0)),
            pl.BlockSpec((1, 400), lambda i: (0, 0)),
            pl.BlockSpec((400, 8), lambda i: (0, 0)),
        ],
        out_specs=pl.BlockSpec((2, _B, 16), lambda i: (0, i, 0)),
        out_shape=jax.ShapeDtypeStruct((2, N, 16), jnp.float32),
    )(s2s, s2t, indeg, b2, W3)


def _final_body(s3_ref, deg_ref, b3_ref, out_ref):
    dinv = lax.rsqrt(deg_ref[...])
    s3 = jnp.concatenate([s3_ref[0][:, 0:4], s3_ref[1][:, 0:4]], axis=1)
    out_ref[...] = jnp.maximum(s3 * dinv + b3_ref[...], 0.0)


def _final(s3, indeg, b3):
    return pl.pallas_call(
        _final_body,
        grid=(N // _B,),
        in_specs=[
            pl.BlockSpec((2, _B, 16), lambda i: (0, i, 0)),
            pl.BlockSpec((_B, 1), lambda i: (i, 0)),
            pl.BlockSpec((1, 8), lambda i: (0, 0)),
        ],
        out_specs=pl.BlockSpec((_B, 8), lambda i: (i, 0)),
        out_shape=jax.ShapeDtypeStruct((N, 8), jnp.float32),
    )(s3, indeg, b3)


_prop64 = _make_propagate(64, 5)
_prop16 = _make_propagate(16, 5)
_l2_kernel = _make_layer2()
_deg_kernel = _make_degree()


def kernel(x, edge_index, W1, b1, W2, b2, W3, b3):
    src = edge_index[0].astype(jnp.int32)
    dst = edge_index[1].astype(jnp.int32)
    gsrc = jnp.stack([src, src + N]).reshape(2, NS, NCH, K)
    gdst = dst.reshape(NS, NCH, K)

    ones16 = jnp.ones((N, 16), jnp.float32)
    deg = _deg_kernel(ones16, gdst)[:N, 0:1]

    u = _scale_x(x, deg)
    s1 = _prop64(u.reshape(2 * N, 64), gsrc, gdst)
    z2s, z2t = _layer12(s1.reshape(2, N, 64), deg, W1, b1.reshape(1, 800), W2)
    s2s, s2t = _l2_kernel(z2s[0:2].reshape(2 * N, 64),
                          z2s[2:4].reshape(2 * N, 64),
                          z2s[4:6].reshape(2 * N, 64),
                          z2t.reshape(2 * N, 16), gsrc, gdst)
    z3 = _layer23(s2s.reshape(6, N, 64), s2t.reshape(2, N, 16), deg,
                  b2.reshape(1, 400), W3)
    s3 = _prop16(z3.reshape(2 * N, 16), gsrc, gdst)
    return _final(s3.reshape(2, N, 16), deg, b3.reshape(1, 8))


# speedup vs baseline: 1.0733x; 1.0733x over previous
import functools

import jax
import jax.numpy as jnp
from jax import lax
from jax.experimental import pallas as pl
from jax.experimental.pallas import tpu as pltpu
from jax.experimental.pallas import tpu_sc as plsc

N = 10000
E = 320000
NS = 16
K = 80
NCH = E // (NS * K)
ROWS_A = 624
TAIL = N - NS * ROWS_A


def _make_propagate(fsc, nbuf):
    mesh = plsc.VectorSubcoreMesh(core_axis_name="c", subcore_axis_name="s")

    @functools.partial(
        pl.kernel,
        mesh=mesh,
        compiler_params=pltpu.CompilerParams(use_tc_tiling_on_sc=False),
        out_type=jax.ShapeDtypeStruct((2 * N, fsc), jnp.float32),
        scratch_types=[
            pltpu.VMEM((NCH, K), jnp.int32),
            pltpu.VMEM((NCH, K), jnp.int32),
        ]
        + [pltpu.VMEM((K, fsc), jnp.float32) for _ in range(nbuf)]
        + [pltpu.VMEM_SHARED((N, fsc), jnp.float32)]
        + [pltpu.SemaphoreType.DMA for _ in range(nbuf)],
    )
    def prop(z_hbm, gsrc_hbm, gdst_hbm, out_hbm, src_v, dst_v, *rest):
        bufs = rest[:nbuf]
        acc = rest[nbuf]
        sems = rest[nbuf + 1:]
        c = lax.axis_index("c")
        s = lax.axis_index("s")
        pltpu.sync_copy(gsrc_hbm.at[c, s], src_v)
        pltpu.sync_copy(gdst_hbm.at[s], dst_v)
        pltpu.sync_copy(z_hbm.at[pl.ds(c * N + s * ROWS_A, ROWS_A)],
                        acc.at[pl.ds(s * ROWS_A, ROWS_A)])

        @pl.when(s == 0)
        def _():
            pltpu.sync_copy(z_hbm.at[pl.ds(c * N + NS * ROWS_A, TAIL)],
                            acc.at[pl.ds(NS * ROWS_A, TAIL)])

        plsc.subcore_barrier()

        def gather(j, b):
            pltpu.make_async_copy(z_hbm.at[src_v.at[j]], bufs[b], sems[b]).start()

        def gwait(b):
            pltpu.make_async_copy(z_hbm.at[src_v.at[0]], bufs[b], sems[b]).wait()

        for b in range(nbuf - 1):
            gather(b, b)

        def body(j0, carry):
            j = j0 * nbuf
            for b in range(nbuf):
                gwait(b)

                @pl.when(j + b + nbuf - 1 < NCH)
                def _():
                    gather(j + b + nbuf - 1, (b + nbuf - 1) % nbuf)

                pltpu.sync_copy(bufs[b], acc.at[dst_v.at[j + b]], add=True)
            return carry

        lax.fori_loop(0, NCH // nbuf, body, 0)
        plsc.subcore_barrier()
        pltpu.sync_copy(acc.at[pl.ds(s * ROWS_A, ROWS_A)],
                        out_hbm.at[pl.ds(c * N + s * ROWS_A, ROWS_A)])

        @pl.when(s == 0)
        def _():
            pltpu.sync_copy(acc.at[pl.ds(NS * ROWS_A, TAIL)],
                            out_hbm.at[pl.ds(c * N + NS * ROWS_A, TAIL)])

    return prop


def _make_layer2():
    mesh = plsc.VectorSubcoreMesh(core_axis_name="c", subcore_axis_name="s")
    nbuf = 5
    nbuf16 = 5

    @functools.partial(
        pl.kernel,
        mesh=mesh,
        compiler_params=pltpu.CompilerParams(use_tc_tiling_on_sc=False),
        out_type=(jax.ShapeDtypeStruct((6 * N, 64), jnp.float32),
                  jax.ShapeDtypeStruct((2 * N, 16), jnp.float32)),
        scratch_types=[
            pltpu.VMEM((NCH, K), jnp.int32),
            pltpu.VMEM((NCH, K), jnp.int32),
        ]
        + [pltpu.VMEM((K, 64), jnp.float32) for _ in range(nbuf)]
        + [pltpu.VMEM((K, 16), jnp.float32) for _ in range(nbuf16)]
        + [pltpu.VMEM_SHARED((N, 64), jnp.float32),
           pltpu.VMEM_SHARED((N, 16), jnp.float32)]
        + [pltpu.SemaphoreType.DMA for _ in range(nbuf)],
    )
    def l2(za_hbm, zb_hbm, zc_hbm, zt_hbm, gsrc_hbm, gdst_hbm,
           out6_hbm, outt_hbm, src_v, dst_v, *rest):
        bufs64 = rest[:nbuf]
        bufs16 = rest[nbuf:nbuf + nbuf16]
        acc64 = rest[nbuf + nbuf16]
        acc16 = rest[nbuf + nbuf16 + 1]
        sems = rest[nbuf + nbuf16 + 2:]
        c = lax.axis_index("c")
        s = lax.axis_index("s")
        pltpu.sync_copy(gsrc_hbm.at[c, s], src_v)
        pltpu.sync_copy(gdst_hbm.at[s], dst_v)

        def round_(q, z_hbm, out_hbm, bufs, acc, obase):
            nb = len(bufs)
            zbase = c * N
            pltpu.sync_copy(z_hbm.at[pl.ds(zbase + s * ROWS_A, ROWS_A)],
                            acc.at[pl.ds(s * ROWS_A, ROWS_A)])

            @pl.when(s == 0)
            def _():
                pltpu.sync_copy(z_hbm.at[pl.ds(zbase + NS * ROWS_A, TAIL)],
                                acc.at[pl.ds(NS * ROWS_A, TAIL)])

            plsc.subcore_barrier()

            def gather(j, b):
                pltpu.make_async_copy(z_hbm.at[src_v.at[j]], bufs[b],
                                      sems[b]).start()

            def gwait(b):
                pltpu.make_async_copy(z_hbm.at[src_v.at[0]], bufs[b],
                                      sems[b]).wait()

            for b in range(nb - 1):
                gather(b, b)

            def body(j0, carry):
                j = j0 * nb
                for b in range(nb):
                    gwait(b)

                    @pl.when(j + b + nb - 1 < NCH)
                    def _():
                        gather(j + b + nb - 1, (b + nb - 1) % nb)

                    pltpu.sync_copy(bufs[b], acc.at[dst_v.at[j + b]], add=True)
                return carry

            lax.fori_loop(0, NCH // nb, body, 0)
            plsc.subcore_barrier()
            pltpu.sync_copy(acc.at[pl.ds(s * ROWS_A, ROWS_A)],
                            out_hbm.at[pl.ds(obase + s * ROWS_A, ROWS_A)])

            @pl.when(s == 0)
            def _():
                pltpu.sync_copy(acc.at[pl.ds(NS * ROWS_A, TAIL)],
                                out_hbm.at[pl.ds(obase + NS * ROWS_A, TAIL)])

        for q, z_hbm in enumerate([za_hbm, zb_hbm, zc_hbm]):
            round_(q, z_hbm, out6_hbm, bufs64, acc64, (2 * q + c) * N)
        round_(3, zt_hbm, outt_hbm, bufs16, acc16, c * N)

    return l2


def _make_degree():
    mesh = plsc.VectorSubcoreMesh(core_axis_name="c", subcore_axis_name="s")

    @functools.partial(
        pl.kernel,
        mesh=mesh,
        compiler_params=pltpu.CompilerParams(use_tc_tiling_on_sc=False),
        out_type=jax.ShapeDtypeStruct((2 * N, 16), jnp.float32),
        scratch_types=[
            pltpu.VMEM((NCH, K), jnp.int32),
            pltpu.VMEM((K, 16), jnp.float32),
            pltpu.VMEM_SHARED((N, 16), jnp.float32),
        ],
    )
    def deg(ones_hbm, gdst_hbm, out_hbm, dst_v, ones_v, acc):
        c = lax.axis_index("c")
        s = lax.axis_index("s")
        pltpu.sync_copy(gdst_hbm.at[s], dst_v)
        pltpu.sync_copy(ones_hbm.at[pl.ds(0, K)], ones_v)
        pltpu.sync_copy(ones_hbm.at[pl.ds(s * ROWS_A, ROWS_A)],
                        acc.at[pl.ds(s * ROWS_A, ROWS_A)])

        @pl.when(s == 0)
        def _():
            pltpu.sync_copy(ones_hbm.at[pl.ds(NS * ROWS_A, TAIL)],
                            acc.at[pl.ds(NS * ROWS_A, TAIL)])

        plsc.subcore_barrier()

        def body(j, carry):
            pltpu.sync_copy(ones_v, acc.at[dst_v.at[j]], add=True)
            return carry

        lax.fori_loop(0, NCH, body, 0)
        plsc.subcore_barrier()
        pltpu.sync_copy(acc.at[pl.ds(s * ROWS_A, ROWS_A)],
                        out_hbm.at[pl.ds(c * N + s * ROWS_A, ROWS_A)])

        @pl.when(s == 0)
        def _():
            pltpu.sync_copy(acc.at[pl.ds(NS * ROWS_A, TAIL)],
                            out_hbm.at[pl.ds(c * N + NS * ROWS_A, TAIL)])

    return deg


_B = 1000


def _scale_body(x_ref, deg_ref, u_ref):
    dinv = lax.rsqrt(deg_ref[...])
    u = x_ref[...] * dinv
    u_ref[0] = u[:, :64]
    u_ref[1] = u[:, 64:]


def _scale_x(x, indeg):
    return pl.pallas_call(
        _scale_body,
        grid=(N // _B,),
        in_specs=[
            pl.BlockSpec((_B, 128), lambda i: (i, 0)),
            pl.BlockSpec((_B, 1), lambda i: (i, 0)),
        ],
        out_specs=pl.BlockSpec((2, _B, 64), lambda i: (0, i, 0)),
        out_shape=jax.ShapeDtypeStruct((2, N, 64), jnp.float32),
    )(x, indeg)


def _layer12_body(s1_ref, deg_ref, w1_ref, b1_ref, w2_ref, z2s_ref, z2t_ref):
    dinv = lax.rsqrt(deg_ref[...])
    s1 = jnp.concatenate([s1_ref[0], s1_ref[1]], axis=1)
    h = jnp.dot(s1 * dinv, w1_ref[...], preferred_element_type=jnp.float32)
    h = jnp.maximum(h + b1_ref[...], 0.0)
    z2 = jnp.dot(h, w2_ref[...], preferred_element_type=jnp.float32) * dinv
    for k in range(6):
        z2s_ref[k] = z2[:, k * 64:(k + 1) * 64]
    pad = jnp.zeros((z2.shape[0], 8), jnp.float32)
    z2t_ref[0] = jnp.concatenate([z2[:, 384:392], pad], axis=1)
    z2t_ref[1] = jnp.concatenate([z2[:, 392:400], pad], axis=1)


def _layer12(s1, indeg, W1, b1, W2):
    return pl.pallas_call(
        _layer12_body,
        grid=(N // _B,),
        in_specs=[
            pl.BlockSpec((2, _B, 64), lambda i: (0, i, 0)),
            pl.BlockSpec((_B, 1), lambda i: (i, 0)),
            pl.BlockSpec((128, 800), lambda i: (0, 0)),
            pl.BlockSpec((1, 800), lambda i: (0, 0)),
            pl.BlockSpec((800, 400), lambda i: (0, 0)),
        ],
        out_specs=[pl.BlockSpec((6, _B, 64), lambda i: (0, i, 0)),
                   pl.BlockSpec((2, _B, 16), lambda i: (0, i, 0))],
        out_shape=[jax.ShapeDtypeStruct((6, N, 64), jnp.float32),
                   jax.ShapeDtypeStruct((2, N, 16), jnp.float32)],
    )(s1, indeg, W1, b1, W2)


def _layer23_body(s2s_ref, s2t_ref, deg_ref, b2_ref, w3_ref, z3_ref):
    dinv = lax.rsqrt(deg_ref[...])
    s2 = jnp.concatenate([s2s_ref[k] for k in range(6)]
                         + [s2t_ref[0][:, 0:8], s2t_ref[1][:, 0:8]], axis=1)
    t2 = jnp.maximum(s2 * dinv + b2_ref[...], 0.0)
    z3 = jnp.dot(t2, w3_ref[...], preferred_element_type=jnp.float32) * dinv
    pad = jnp.zeros((z3.shape[0], 12), jnp.float32)
    z3_ref[0] = jnp.concatenate([z3[:, :4], pad], axis=1)
    z3_ref[1] = jnp.concatenate([z3[:, 4:], pad], axis=1)


def _layer23(s2s, s2t, indeg, b2, W3):
    return pl.pallas_call(
        _layer23_body,
        grid=(N // _B,),
        in_specs=[
            pl.BlockSpec((6, _B, 64), lambda i: (0, i, 0)),
            pl.BlockSpec((2, _B, 16), lambda i: (0, i, 0)),
            pl.BlockSpec((_B, 1), lambda i: (i, 0)),
            pl.BlockSpec((1, 400), lambda i: (0, 0)),
            pl.BlockSpec((400, 8), lambda i: (0, 0)),
        ],
        out_specs=pl.BlockSpec((2, _B, 16), lambda i: (0, i, 0)),
        out_shape=jax.ShapeDtypeStruct((2, N, 16), jnp.float32),
    )(s2s, s2t, indeg, b2, W3)


def _final_body(s3_ref, deg_ref, b3_ref, out_ref):
    dinv = lax.rsqrt(deg_ref[...])
    s3 = jnp.concatenate([s3_ref[0][:, 0:4], s3_ref[1][:, 0:4]], axis=1)
    out_ref[...] = jnp.maximum(s3 * dinv + b3_ref[...], 0.0)


def _final(s3, indeg, b3):
    return pl.pallas_call(
        _final_body,
        grid=(N // _B,),
        in_specs=[
            pl.BlockSpec((2, _B, 16), lambda i: (0, i, 0)),
            pl.BlockSpec((_B, 1), lambda i: (i, 0)),
            pl.BlockSpec((1, 8), lambda i: (0, 0)),
        ],
        out_specs=pl.BlockSpec((_B, 8), lambda i: (i, 0)),
        out_shape=jax.ShapeDtypeStruct((N, 8), jnp.float32),
    )(s3, indeg, b3)


_prop64 = _make_propagate(64, 5)
_prop16 = _make_propagate(16, 5)
_l2_kernel = _make_layer2()
_deg_kernel = _make_degree()


def kernel(x, edge_index, W1, b1, W2, b2, W3, b3):
    src = edge_index[0].astype(jnp.int32)
    dst = edge_index[1].astype(jnp.int32)
    gsrc = jnp.stack([src, src + N]).reshape(2, NS, NCH, K)
    gdst = dst.reshape(NS, NCH, K)

    ones16 = jnp.ones((N, 16), jnp.float32)
    deg = _deg_kernel(ones16, gdst)[:N, 0:1]

    u = _scale_x(x, deg)
    s1 = _prop64(u.reshape(2 * N, 64), gsrc, gdst)
    z2s, z2t = _layer12(s1.reshape(2, N, 64), deg, W1, b1.reshape(1, 800), W2)
    s2s, s2t = _l2_kernel(z2s[0:2].reshape(2 * N, 64),
                          z2s[2:4].reshape(2 * N, 64),
                          z2s[4:6].reshape(2 * N, 64),
                          z2t.reshape(2 * N, 16), gsrc, gdst)
    z3 = _layer23(s2s.reshape(6, N, 64), s2t.reshape(2, N, 16), deg,
                  b2.reshape(1, 400), W3)
    s3 = _prop16(z3.reshape(2 * N, 16), gsrc, gdst)
    return _final(s3.reshape(2, N, 16), deg, b3.reshape(1, 8))


# speedup vs baseline: 1.0749x; 1.0015x over previous
import functools

import jax
import jax.numpy as jnp
from jax import lax
from jax.experimental import pallas as pl
from jax.experimental.pallas import tpu as pltpu
from jax.experimental.pallas import tpu_sc as plsc

N = 10000
E = 320000
NS = 16
K = 80
NCH = E // (NS * K)
ROWS_A = 624
TAIL = N - NS * ROWS_A


def _make_propagate(fsc, nbuf):
    mesh = plsc.VectorSubcoreMesh(core_axis_name="c", subcore_axis_name="s")

    @functools.partial(
        pl.kernel,
        mesh=mesh,
        compiler_params=pltpu.CompilerParams(use_tc_tiling_on_sc=False),
        out_type=jax.ShapeDtypeStruct((2 * N, fsc), jnp.float32),
        scratch_types=[
            pltpu.VMEM((NCH, K), jnp.int32),
            pltpu.VMEM((NCH, K), jnp.int32),
        ]
        + [pltpu.VMEM((K, fsc), jnp.float32) for _ in range(nbuf)]
        + [pltpu.VMEM_SHARED((N, fsc), jnp.float32)]
        + [pltpu.SemaphoreType.DMA for _ in range(nbuf)],
    )
    def prop(z_hbm, gsrc_hbm, gdst_hbm, out_hbm, src_v, dst_v, *rest):
        bufs = rest[:nbuf]
        acc = rest[nbuf]
        sems = rest[nbuf + 1:]
        c = lax.axis_index("c")
        s = lax.axis_index("s")
        pltpu.sync_copy(gsrc_hbm.at[s], src_v)
        pltpu.sync_copy(gdst_hbm.at[s], dst_v)
        base = c * N

        def _off(j, carry):
            for k in range(K // 16):
                src_v[j, pl.ds(k * 16, 16)] = src_v[j, pl.ds(k * 16, 16)] + base
            return carry

        lax.fori_loop(0, NCH, _off, 0)
        pltpu.sync_copy(z_hbm.at[pl.ds(c * N + s * ROWS_A, ROWS_A)],
                        acc.at[pl.ds(s * ROWS_A, ROWS_A)])

        @pl.when(s == 0)
        def _():
            pltpu.sync_copy(z_hbm.at[pl.ds(c * N + NS * ROWS_A, TAIL)],
                            acc.at[pl.ds(NS * ROWS_A, TAIL)])

        plsc.subcore_barrier()

        def gather(j, b):
            pltpu.make_async_copy(z_hbm.at[src_v.at[j]], bufs[b], sems[b]).start()

        def gwait(b):
            pltpu.make_async_copy(z_hbm.at[src_v.at[0]], bufs[b], sems[b]).wait()

        for b in range(nbuf - 1):
            gather(b, b)

        def body(j0, carry):
            j = j0 * nbuf
            for b in range(nbuf):
                gwait(b)

                @pl.when(j + b + nbuf - 1 < NCH)
                def _():
                    gather(j + b + nbuf - 1, (b + nbuf - 1) % nbuf)

                pltpu.sync_copy(bufs[b], acc.at[dst_v.at[j + b]], add=True)
            return carry

        lax.fori_loop(0, NCH // nbuf, body, 0)
        plsc.subcore_barrier()
        pltpu.sync_copy(acc.at[pl.ds(s * ROWS_A, ROWS_A)],
                        out_hbm.at[pl.ds(c * N + s * ROWS_A, ROWS_A)])

        @pl.when(s == 0)
        def _():
            pltpu.sync_copy(acc.at[pl.ds(NS * ROWS_A, TAIL)],
                            out_hbm.at[pl.ds(c * N + NS * ROWS_A, TAIL)])

    return prop


def _make_layer2():
    mesh = plsc.VectorSubcoreMesh(core_axis_name="c", subcore_axis_name="s")
    nbuf = 5
    nbuf16 = 5

    @functools.partial(
        pl.kernel,
        mesh=mesh,
        compiler_params=pltpu.CompilerParams(use_tc_tiling_on_sc=False),
        out_type=(jax.ShapeDtypeStruct((6 * N, 64), jnp.float32),
                  jax.ShapeDtypeStruct((2 * N, 16), jnp.float32)),
        scratch_types=[
            pltpu.VMEM((NCH, K), jnp.int32),
            pltpu.VMEM((NCH, K), jnp.int32),
        ]
        + [pltpu.VMEM((K, 64), jnp.float32) for _ in range(nbuf)]
        + [pltpu.VMEM((K, 16), jnp.float32) for _ in range(nbuf16)]
        + [pltpu.VMEM_SHARED((N, 64), jnp.float32),
           pltpu.VMEM_SHARED((N, 16), jnp.float32)]
        + [pltpu.SemaphoreType.DMA for _ in range(nbuf)],
    )
    def l2(za_hbm, zb_hbm, zc_hbm, zt_hbm, gsrc_hbm, gdst_hbm,
           out6_hbm, outt_hbm, src_v, dst_v, *rest):
        bufs64 = rest[:nbuf]
        bufs16 = rest[nbuf:nbuf + nbuf16]
        acc64 = rest[nbuf + nbuf16]
        acc16 = rest[nbuf + nbuf16 + 1]
        sems = rest[nbuf + nbuf16 + 2:]
        c = lax.axis_index("c")
        s = lax.axis_index("s")
        pltpu.sync_copy(gsrc_hbm.at[s], src_v)
        pltpu.sync_copy(gdst_hbm.at[s], dst_v)
        base = c * N

        def _off(j, carry):
            for k in range(K // 16):
                src_v[j, pl.ds(k * 16, 16)] = src_v[j, pl.ds(k * 16, 16)] + base
            return carry

        lax.fori_loop(0, NCH, _off, 0)

        def round_(q, z_hbm, out_hbm, bufs, acc, obase):
            nb = len(bufs)
            zbase = c * N
            pltpu.sync_copy(z_hbm.at[pl.ds(zbase + s * ROWS_A, ROWS_A)],
                            acc.at[pl.ds(s * ROWS_A, ROWS_A)])

            @pl.when(s == 0)
            def _():
                pltpu.sync_copy(z_hbm.at[pl.ds(zbase + NS * ROWS_A, TAIL)],
                                acc.at[pl.ds(NS * ROWS_A, TAIL)])

            plsc.subcore_barrier()

            def gather(j, b):
                pltpu.make_async_copy(z_hbm.at[src_v.at[j]], bufs[b],
                                      sems[b]).start()

            def gwait(b):
                pltpu.make_async_copy(z_hbm.at[src_v.at[0]], bufs[b],
                                      sems[b]).wait()

            for b in range(nb - 1):
                gather(b, b)

            def body(j0, carry):
                j = j0 * nb
                for b in range(nb):
                    gwait(b)

                    @pl.when(j + b + nb - 1 < NCH)
                    def _():
                        gather(j + b + nb - 1, (b + nb - 1) % nb)

                    pltpu.sync_copy(bufs[b], acc.at[dst_v.at[j + b]], add=True)
                return carry

            lax.fori_loop(0, NCH // nb, body, 0)
            plsc.subcore_barrier()
            pltpu.sync_copy(acc.at[pl.ds(s * ROWS_A, ROWS_A)],
                            out_hbm.at[pl.ds(obase + s * ROWS_A, ROWS_A)])

            @pl.when(s == 0)
            def _():
                pltpu.sync_copy(acc.at[pl.ds(NS * ROWS_A, TAIL)],
                                out_hbm.at[pl.ds(obase + NS * ROWS_A, TAIL)])

        for q, z_hbm in enumerate([za_hbm, zb_hbm, zc_hbm]):
            round_(q, z_hbm, out6_hbm, bufs64, acc64, (2 * q + c) * N)
        round_(3, zt_hbm, outt_hbm, bufs16, acc16, c * N)

    return l2


def _make_degree():
    mesh = plsc.VectorSubcoreMesh(core_axis_name="c", subcore_axis_name="s")

    @functools.partial(
        pl.kernel,
        mesh=mesh,
        compiler_params=pltpu.CompilerParams(use_tc_tiling_on_sc=False),
        out_type=jax.ShapeDtypeStruct((2 * N, 16), jnp.float32),
        scratch_types=[
            pltpu.VMEM((NCH, K), jnp.int32),
            pltpu.VMEM((K, 16), jnp.float32),
            pltpu.VMEM_SHARED((N, 16), jnp.float32),
        ],
    )
    def deg(ones_hbm, gdst_hbm, out_hbm, dst_v, ones_v, acc):
        c = lax.axis_index("c")
        s = lax.axis_index("s")
        pltpu.sync_copy(gdst_hbm.at[s], dst_v)
        pltpu.sync_copy(ones_hbm.at[pl.ds(0, K)], ones_v)
        pltpu.sync_copy(ones_hbm.at[pl.ds(s * ROWS_A, ROWS_A)],
                        acc.at[pl.ds(s * ROWS_A, ROWS_A)])

        @pl.when(s == 0)
        def _():
            pltpu.sync_copy(ones_hbm.at[pl.ds(NS * ROWS_A, TAIL)],
                            acc.at[pl.ds(NS * ROWS_A, TAIL)])

        plsc.subcore_barrier()

        def body(j, carry):
            pltpu.sync_copy(ones_v, acc.at[dst_v.at[j]], add=True)
            return carry

        lax.fori_loop(0, NCH, body, 0)
        plsc.subcore_barrier()
        pltpu.sync_copy(acc.at[pl.ds(s * ROWS_A, ROWS_A)],
                        out_hbm.at[pl.ds(c * N + s * ROWS_A, ROWS_A)])

        @pl.when(s == 0)
        def _():
            pltpu.sync_copy(acc.at[pl.ds(NS * ROWS_A, TAIL)],
                            out_hbm.at[pl.ds(c * N + NS * ROWS_A, TAIL)])

    return deg


_B = 1000


def _scale_body(x_ref, deg_ref, u_ref):
    dinv = lax.rsqrt(deg_ref[...])
    u = x_ref[...] * dinv
    u_ref[0] = u[:, :64]
    u_ref[1] = u[:, 64:]


def _scale_x(x, indeg):
    return pl.pallas_call(
        _scale_body,
        grid=(N // _B,),
        in_specs=[
            pl.BlockSpec((_B, 128), lambda i: (i, 0)),
            pl.BlockSpec((_B, 1), lambda i: (i, 0)),
        ],
        out_specs=pl.BlockSpec((2, _B, 64), lambda i: (0, i, 0)),
        out_shape=jax.ShapeDtypeStruct((2, N, 64), jnp.float32),
    )(x, indeg)


def _layer12_body(s1_ref, deg_ref, w1_ref, b1_ref, w2_ref, z2s_ref, z2t_ref):
    dinv = lax.rsqrt(deg_ref[...])
    s1 = jnp.concatenate([s1_ref[0], s1_ref[1]], axis=1)
    h = jnp.dot(s1 * dinv, w1_ref[...], preferred_element_type=jnp.float32)
    h = jnp.maximum(h + b1_ref[...], 0.0)
    z2 = jnp.dot(h, w2_ref[...], preferred_element_type=jnp.float32) * dinv
    for k in range(6):
        z2s_ref[k] = z2[:, k * 64:(k + 1) * 64]
    pad = jnp.zeros((z2.shape[0], 8), jnp.float32)
    z2t_ref[0] = jnp.concatenate([z2[:, 384:392], pad], axis=1)
    z2t_ref[1] = jnp.concatenate([z2[:, 392:400], pad], axis=1)


def _layer12(s1, indeg, W1, b1, W2):
    return pl.pallas_call(
        _layer12_body,
        grid=(N // _B,),
        in_specs=[
            pl.BlockSpec((2, _B, 64), lambda i: (0, i, 0)),
            pl.BlockSpec((_B, 1), lambda i: (i, 0)),
            pl.BlockSpec((128, 800), lambda i: (0, 0)),
            pl.BlockSpec((1, 800), lambda i: (0, 0)),
            pl.BlockSpec((800, 400), lambda i: (0, 0)),
        ],
        out_specs=[pl.BlockSpec((6, _B, 64), lambda i: (0, i, 0)),
                   pl.BlockSpec((2, _B, 16), lambda i: (0, i, 0))],
        out_shape=[jax.ShapeDtypeStruct((6, N, 64), jnp.float32),
                   jax.ShapeDtypeStruct((2, N, 16), jnp.float32)],
    )(s1, indeg, W1, b1, W2)


def _layer23_body(s2s_ref, s2t_ref, deg_ref, b2_ref, w3_ref, z3_ref):
    dinv = lax.rsqrt(deg_ref[...])
    s2 = jnp.concatenate([s2s_ref[k] for k in range(6)]
                         + [s2t_ref[0][:, 0:8], s2t_ref[1][:, 0:8]], axis=1)
    t2 = jnp.maximum(s2 * dinv + b2_ref[...], 0.0)
    z3 = jnp.dot(t2, w3_ref[...], preferred_element_type=jnp.float32) * dinv
    pad = jnp.zeros((z3.shape[0], 12), jnp.float32)
    z3_ref[0] = jnp.concatenate([z3[:, :4], pad], axis=1)
    z3_ref[1] = jnp.concatenate([z3[:, 4:], pad], axis=1)


def _layer23(s2s, s2t, indeg, b2, W3):
    return pl.pallas_call(
        _layer23_body,
        grid=(N // _B,),
        in_specs=[
            pl.BlockSpec((6, _B, 64), lambda i: (0, i, 0)),
            pl.BlockSpec((2, _B, 16), lambda i: (0, i, 0)),
            pl.BlockSpec((_B, 1), lambda i: (i, 0)),
            pl.BlockSpec((1, 400), lambda i: (0, 0)),
            pl.BlockSpec((400, 8), lambda i: (0, 0)),
        ],
        out_specs=pl.BlockSpec((2, _B, 16), lambda i: (0, i, 0)),
        out_shape=jax.ShapeDtypeStruct((2, N, 16), jnp.float32),
    )(s2s, s2t, indeg, b2, W3)


def _final_body(s3_ref, deg_ref, b3_ref, out_ref):
    dinv = lax.rsqrt(deg_ref[...])
    s3 = jnp.concatenate([s3_ref[0][:, 0:4], s3_ref[1][:, 0:4]], axis=1)
    out_ref[...] = jnp.maximum(s3 * dinv + b3_ref[...], 0.0)


def _final(s3, indeg, b3):
    return pl.pallas_call(
        _final_body,
        grid=(N // _B,),
        in_specs=[
            pl.BlockSpec((2, _B, 16), lambda i: (0, i, 0)),
            pl.BlockSpec((_B, 1), lambda i: (i, 0)),
            pl.BlockSpec((1, 8), lambda i: (0, 0)),
        ],
        out_specs=pl.BlockSpec((_B, 8), lambda i: (i, 0)),
        out_shape=jax.ShapeDtypeStruct((N, 8), jnp.float32),
    )(s3, indeg, b3)


_prop64 = _make_propagate(64, 5)
_prop16 = _make_propagate(16, 5)
_l2_kernel = _make_layer2()
_deg_kernel = _make_degree()


def kernel(x, edge_index, W1, b1, W2, b2, W3, b3):
    src = edge_index[0].astype(jnp.int32)
    dst = edge_index[1].astype(jnp.int32)
    gsrc = src.reshape(NS, NCH, K)
    gdst = dst.reshape(NS, NCH, K)

    ones16 = jnp.ones((N, 16), jnp.float32)
    deg = _deg_kernel(ones16, gdst)[:N, 0:1]

    u = _scale_x(x, deg)
    s1 = _prop64(u.reshape(2 * N, 64), gsrc, gdst)
    z2s, z2t = _layer12(s1.reshape(2, N, 64), deg, W1, b1.reshape(1, 800), W2)
    s2s, s2t = _l2_kernel(z2s[0:2].reshape(2 * N, 64),
                          z2s[2:4].reshape(2 * N, 64),
                          z2s[4:6].reshape(2 * N, 64),
                          z2t.reshape(2 * N, 16), gsrc, gdst)
    z3 = _layer23(s2s.reshape(6, N, 64), s2t.reshape(2, N, 16), deg,
                  b2.reshape(1, 400), W3)
    s3 = _prop16(z3.reshape(2 * N, 16), gsrc, gdst)
    return _final(s3.reshape(2, N, 16), deg, b3.reshape(1, 8))
